# baseline (speedup 1.0000x reference)
import functools

import jax
import jax.numpy as jnp
from jax import lax
from jax.experimental import pallas as pl
from jax.experimental.pallas import tpu as pltpu
from jax.experimental.pallas import tpu_sc as plsc

B = 16384
D = 768
NC = 2
NS = 16
NW = NC * NS
BPW = B // NW
CHUNK = 32
NBUF = 4
NCHUNK = BPW // CHUNK


def _make_gather(V):
    mesh = plsc.VectorSubcoreMesh(core_axis_name="c", subcore_axis_name="s")

    @functools.partial(
        pl.kernel,
        mesh=mesh,
        out_type=jax.ShapeDtypeStruct((B, 1, D), jnp.float32),
        scratch_types=[
            pltpu.VMEM((NCHUNK, CHUNK), jnp.int32),
            pltpu.VMEM((NBUF, CHUNK, D), jnp.float32),
        ] + [pltpu.SemaphoreType.DMA] * (2 * NBUF),
    )
    def gather_kernel(table_hbm, idx_hbm, out_hbm, idx_v, rows_v, *sems):
        wid = lax.axis_index("s") * NC + lax.axis_index("c")
        base = wid * BPW
        gsem = sems[:NBUF]
        osem = sems[NBUF:]
        for c in range(NCHUNK):
            pltpu.sync_copy(idx_hbm.at[pl.ds(base + c * CHUNK, CHUNK)],
                            idx_v.at[c])

        gh = [None] * NBUF
        oh = [None] * NBUF

        def start_gather(c):
            buf = c % NBUF
            if oh[buf] is not None:
                oh[buf].wait()
            gh[buf] = pltpu.async_copy(table_hbm.at[idx_v.at[c]],
                                       rows_v.at[buf], gsem[buf])

        def start_out(c):
            buf = c % NBUF
            gh[buf].wait()
            oh[buf] = pltpu.async_copy(
                rows_v.at[buf],
                out_hbm.at[pl.ds(base + c * CHUNK, CHUNK), 0],
                osem[buf])

        for c in range(min(NBUF, NCHUNK)):
            start_gather(c)
        for c in range(NCHUNK):
            start_out(c)
            if c + NBUF < NCHUNK:
                start_gather(c + NBUF)
        for buf in range(NBUF):
            if oh[buf] is not None:
                oh[buf].wait()

    return gather_kernel


def kernel(ids, table):
    ids = ids.astype(jnp.int32)
    return _make_gather(table.shape[0])(table, ids)


# speedup vs baseline: 1.1107x; 1.1107x over previous
import functools

import jax
import jax.numpy as jnp
from jax import lax
from jax.experimental import pallas as pl
from jax.experimental.pallas import tpu as pltpu
from jax.experimental.pallas import tpu_sc as plsc

B = 16384
D = 768
NC = 2
NS = 16
NW = NC * NS
BPW = B // NW
CHUNK = 64
NBUF = 2
NCHUNK = BPW // CHUNK


def _make_gather(V):
    mesh = plsc.VectorSubcoreMesh(core_axis_name="c", subcore_axis_name="s")

    @functools.partial(
        pl.kernel,
        mesh=mesh,
        out_type=jax.ShapeDtypeStruct((B, 1, D), jnp.float32),
        scratch_types=[
            pltpu.VMEM((BPW,), jnp.int32),
            pltpu.VMEM((NBUF, CHUNK, D), jnp.float32),
        ] + [pltpu.SemaphoreType.DMA] * (2 * NBUF),
    )
    def gather_kernel(table_hbm, idx_hbm, out_hbm, idx_v, rows_v, *sems):
        wid = lax.axis_index("s") * NC + lax.axis_index("c")
        base = wid * BPW
        gsem = sems[:NBUF]
        osem = sems[NBUF:]
        pltpu.sync_copy(idx_hbm.at[pl.ds(base, BPW)], idx_v)

        gh = [None] * NBUF
        oh = [None] * NBUF

        def start_gather(c):
            buf = c % NBUF
            if oh[buf] is not None:
                oh[buf].wait()
            gh[buf] = pltpu.async_copy(
                table_hbm.at[idx_v.at[pl.ds(c * CHUNK, CHUNK)]],
                rows_v.at[buf], gsem[buf])

        def start_out(c):
            buf = c % NBUF
            gh[buf].wait()
            oh[buf] = pltpu.async_copy(
                rows_v.at[buf],
                out_hbm.at[pl.ds(base + c * CHUNK, CHUNK), 0],
                osem[buf])

        for c in range(min(NBUF, NCHUNK)):
            start_gather(c)
        for c in range(NCHUNK):
            start_out(c)
            if c + NBUF < NCHUNK:
                start_gather(c + NBUF)
        for buf in range(NBUF):
            if oh[buf] is not None:
                oh[buf].wait()

    return gather_kernel


def kernel(ids, table):
    ids = ids.astype(jnp.int32)
    return _make_gather(table.shape[0])(table, ids)
